# Initial kernel scaffold; baseline (speedup 1.0000x reference)
import jax
import jax.numpy as jnp
from jax.experimental import pallas as pl


def kernel(feature_values, feature_idx, seq_lens, weights_first_order):
    raise NotImplementedError("write your pallas kernel here")


# speedup vs baseline: 65.1694x; 65.1694x over previous
import functools

import jax
import jax.numpy as jnp
from jax import lax
from jax.experimental import pallas as pl
from jax.experimental.pallas import tpu as pltpu
from jax.experimental.pallas import tpu_sc as plsc

FIELDS = 26
BATCH = 4096
MAXLEN = 20
ROWS = FIELDS * BATCH

NC, NS, LANES = 2, 16, 16
NW = NC * NS
ROWS_PER_W = ROWS // NW
CHUNK = 128
GCHUNK = 128
N_GATHERS = CHUNK * MAXLEN // GCHUNK
N_CHUNKS = ROWS_PER_W // CHUNK


def _sc_body(vals_hbm, idx_hbm, lens_hbm, table_hbm, out_hbm,
             idx_v, vals_v, gath_v, lens_v, out_v, sem, gsem):
    wid = lax.axis_index("s") * NC + lax.axis_index("c")

    def chunk_body(ci, _):
        r0 = wid * ROWS_PER_W + ci * CHUNK
        i0 = r0 * MAXLEN
        pltpu.sync_copy(idx_hbm.at[pl.ds(i0, CHUNK * MAXLEN)], idx_v)
        pltpu.sync_copy(vals_hbm.at[pl.ds(i0, CHUNK * MAXLEN)], vals_v)
        pltpu.sync_copy(lens_hbm.at[pl.ds(r0, CHUNK)], lens_v)
        copies = []
        for j in range(N_GATHERS):
            sl = pl.ds(j * GCHUNK, GCHUNK)
            copies.append(
                pltpu.async_copy(table_hbm.at[idx_v.at[sl]], gath_v.at[sl], gsem))
        for cp in copies:
            cp.wait()
        for g in range(CHUNK // LANES):
            base = g * LANES * MAXLEN
            rvec = lax.iota(jnp.int32, LANES) * MAXLEN + base
            acc = jnp.zeros((LANES,), jnp.float32)
            for l in range(MAXLEN):
                ivec = rvec + l
                v = plsc.load_gather(vals_v, [ivec])
                w = plsc.load_gather(gath_v, [ivec])
                acc = acc + v * w
            sl16 = pl.ds(g * LANES, LANES)
            out_v[sl16] = acc / lens_v[sl16]
        pltpu.sync_copy(out_v, out_hbm.at[pl.ds(r0, CHUNK)])
        return ()

    lax.fori_loop(0, N_CHUNKS, chunk_body, ())


@jax.jit
def _sc_call(vals, idx, lens, table):
    mesh = plsc.VectorSubcoreMesh(
        core_axis_name="c", subcore_axis_name="s",
        num_cores=NC, num_subcores=NS)
    return pl.kernel(
        _sc_body,
        out_type=jax.ShapeDtypeStruct((ROWS,), jnp.float32),
        mesh=mesh,
        compiler_params=pltpu.CompilerParams(needs_layout_passes=False),
        scratch_types=[
            pltpu.VMEM((CHUNK * MAXLEN,), jnp.int32),
            pltpu.VMEM((CHUNK * MAXLEN,), jnp.float32),
            pltpu.VMEM((CHUNK * MAXLEN,), jnp.float32),
            pltpu.VMEM((CHUNK,), jnp.float32),
            pltpu.VMEM((CHUNK,), jnp.float32),
            pltpu.SemaphoreType.DMA,
            pltpu.SemaphoreType.DMA,
        ],
    )(vals, idx, lens, table)


def kernel(feature_values, feature_idx, seq_lens, weights_first_order):
    vals = feature_values.reshape(-1)
    idx = feature_idx.astype(jnp.int32).reshape(-1)
    lens = seq_lens.astype(jnp.float32).T.reshape(-1)
    table = weights_first_order.reshape(-1)
    out = _sc_call(vals, idx, lens, table)
    return out.reshape(FIELDS, BATCH).T


# speedup vs baseline: 81.6229x; 1.2525x over previous
import jax
import jax.numpy as jnp
from jax import lax
from jax.experimental import pallas as pl
from jax.experimental.pallas import tpu as pltpu
from jax.experimental.pallas import tpu_sc as plsc

FIELDS = 26
BATCH = 4096
MAXLEN = 20
ROWS = FIELDS * BATCH

NC, NS, LANES = 2, 16, 16
NW = NC * NS
ROWS_PER_W = ROWS // NW
CHUNK = 128
IN = CHUNK * MAXLEN
GCHUNK = 128
N_GATHERS = IN // GCHUNK
N_CHUNKS = ROWS_PER_W // CHUNK


def _sc_body(vals_hbm, idx_hbm, lens_hbm, table_hbm, out_hbm,
             idx_v0, idx_v1, vals_v0, vals_v1, gath_v0, gath_v1,
             lens_v0, lens_v1, out_v0, out_v1,
             isem0, isem1, gsem0, gsem1, osem0, osem1):
    row0 = (lax.axis_index("s") * NC + lax.axis_index("c")) * ROWS_PER_W
    idx_v = (idx_v0, idx_v1)
    vals_v = (vals_v0, vals_v1)
    gath_v = (gath_v0, gath_v1)
    lens_v = (lens_v0, lens_v1)
    out_v = (out_v0, out_v1)
    isem = (isem0, isem1)
    gsem = (gsem0, gsem1)
    osem = (osem0, osem1)

    def in_descs(ci, s):
        r0 = row0 + ci * CHUNK
        return (
            pltpu.make_async_copy(idx_hbm.at[pl.ds(r0 * MAXLEN, IN)], idx_v[s], isem[s]),
            pltpu.make_async_copy(vals_hbm.at[pl.ds(r0 * MAXLEN, IN)], vals_v[s], isem[s]),
            pltpu.make_async_copy(lens_hbm.at[pl.ds(r0, CHUNK)], lens_v[s], isem[s]),
        )

    def fire_in(ci, s):
        for d in in_descs(ci, s):
            d.start()

    def wait_in(ci, s):
        for d in in_descs(ci, s):
            d.wait()

    def g_descs(s):
        out = []
        for j in range(N_GATHERS):
            sl = pl.ds(j * GCHUNK, GCHUNK)
            out.append(pltpu.make_async_copy(
                table_hbm.at[idx_v[s].at[sl]], gath_v[s].at[sl], gsem[s]))
        return out

    def fire_g(s):
        for d in g_descs(s):
            d.start()

    def drain_g(s):
        for d in g_descs(s):
            d.wait()

    def out_desc(ci, s):
        r0 = row0 + ci * CHUNK
        return pltpu.make_async_copy(out_v[s], out_hbm.at[pl.ds(r0, CHUNK)], osem[s])

    def compute(ci, s):
        vv, gv, lv, ov = vals_v[s], gath_v[s], lens_v[s], out_v[s]

        def gbody(g, _):
            rvec = lax.iota(jnp.int32, LANES) * MAXLEN + g * (LANES * MAXLEN)
            acc = jnp.zeros((LANES,), jnp.float32)
            for l in range(MAXLEN):
                ivec = rvec + l
                acc = acc + plsc.load_gather(vv, [ivec]) * plsc.load_gather(gv, [ivec])
            sl = pl.ds(g * LANES, LANES)
            ov[sl] = acc / lv[sl]
            return ()

        lax.fori_loop(0, CHUNK // LANES, gbody, ())
        out_desc(ci, s).start()

    fire_in(0, 0)
    fire_in(1, 1)
    wait_in(0, 0)
    fire_g(0)

    def step(c, s, first, next_g, next_in):
        n = 1 - s
        if next_g:
            wait_in(c + 1, n)
            fire_g(n)
        drain_g(s)
        if not first:
            out_desc(c - 2, s).wait()
        compute(c, s)
        if next_in:
            fire_in(c + 2, s)

    step(0, 0, True, True, True)
    step(1, 1, True, True, True)

    def pair(k, _):
        c = 2 * k
        step(c, 0, False, True, True)
        step(c + 1, 1, False, True, True)
        return ()

    lax.fori_loop(1, (N_CHUNKS - 2) // 2, pair, ())
    step(N_CHUNKS - 2, 0, False, True, False)
    step(N_CHUNKS - 1, 1, False, False, False)
    out_desc(N_CHUNKS - 2, 0).wait()
    out_desc(N_CHUNKS - 1, 1).wait()


@jax.jit
def _sc_call(vals, idx, lens, table):
    mesh = plsc.VectorSubcoreMesh(
        core_axis_name="c", subcore_axis_name="s",
        num_cores=NC, num_subcores=NS)
    dv = lambda shape, dt: pltpu.VMEM(shape, dt)
    return pl.kernel(
        _sc_body,
        out_type=jax.ShapeDtypeStruct((ROWS,), jnp.float32),
        mesh=mesh,
        compiler_params=pltpu.CompilerParams(needs_layout_passes=False),
        scratch_types=[
            dv((IN,), jnp.int32), dv((IN,), jnp.int32),
            dv((IN,), jnp.float32), dv((IN,), jnp.float32),
            dv((IN,), jnp.float32), dv((IN,), jnp.float32),
            dv((CHUNK,), jnp.float32), dv((CHUNK,), jnp.float32),
            dv((CHUNK,), jnp.float32), dv((CHUNK,), jnp.float32),
            pltpu.SemaphoreType.DMA, pltpu.SemaphoreType.DMA,
            pltpu.SemaphoreType.DMA, pltpu.SemaphoreType.DMA,
            pltpu.SemaphoreType.DMA, pltpu.SemaphoreType.DMA,
        ],
    )(vals, idx, lens, table)


def kernel(feature_values, feature_idx, seq_lens, weights_first_order):
    vals = feature_values.reshape(-1)
    idx = feature_idx.astype(jnp.int32).reshape(-1)
    lens = seq_lens.astype(jnp.float32).T.reshape(-1)
    table = weights_first_order.reshape(-1)
    out = _sc_call(vals, idx, lens, table)
    return out.reshape(FIELDS, BATCH).T


# speedup vs baseline: 99.5164x; 1.2192x over previous
import jax
import jax.numpy as jnp
from jax import lax
from jax.experimental import pallas as pl
from jax.experimental.pallas import tpu as pltpu
from jax.experimental.pallas import tpu_sc as plsc

FIELDS = 26
BATCH = 4096
MAXLEN = 20
ROWS = FIELDS * BATCH

NC, NS, LANES = 2, 16, 16
NW = NC * NS
ROWS_PER_W = ROWS // NW
CHUNK = 128
IN = CHUNK * MAXLEN
GCHUNK = 128
N_GATHERS = IN // GCHUNK
N_CHUNKS = ROWS_PER_W // CHUNK


def _sc_body(vals_hbm, idx_hbm, lens_hbm, table_hbm, out_hbm,
             table_sh,
             idx_v0, idx_v1, vals_v0, vals_v1, gath_v0, gath_v1,
             lens_v0, lens_v1, out_v0, out_v1,
             isem0, isem1, gsem0, gsem1, osem0, osem1):
    sid = lax.axis_index("s")
    row0 = (sid * NC + lax.axis_index("c")) * ROWS_PER_W

    @pl.when(sid == 0)
    def _():
        pltpu.sync_copy(table_hbm, table_sh)

    plsc.subcore_barrier()
    idx_v = (idx_v0, idx_v1)
    vals_v = (vals_v0, vals_v1)
    gath_v = (gath_v0, gath_v1)
    lens_v = (lens_v0, lens_v1)
    out_v = (out_v0, out_v1)
    isem = (isem0, isem1)
    gsem = (gsem0, gsem1)
    osem = (osem0, osem1)

    def in_descs(ci, s):
        r0 = row0 + ci * CHUNK
        return (
            pltpu.make_async_copy(idx_hbm.at[pl.ds(r0 * MAXLEN, IN)], idx_v[s], isem[s]),
            pltpu.make_async_copy(vals_hbm.at[pl.ds(r0 * MAXLEN, IN)], vals_v[s], isem[s]),
            pltpu.make_async_copy(lens_hbm.at[pl.ds(r0, CHUNK)], lens_v[s], isem[s]),
        )

    def fire_in(ci, s):
        for d in in_descs(ci, s):
            d.start()

    def wait_in(ci, s):
        for d in in_descs(ci, s):
            d.wait()

    def g_descs(s):
        out = []
        for j in range(N_GATHERS):
            sl = pl.ds(j * GCHUNK, GCHUNK)
            out.append(pltpu.make_async_copy(
                table_sh.at[idx_v[s].at[sl]], gath_v[s].at[sl], gsem[s]))
        return out

    def fire_g(s):
        for d in g_descs(s):
            d.start()

    def drain_g(s):
        for d in g_descs(s):
            d.wait()

    def out_desc(ci, s):
        r0 = row0 + ci * CHUNK
        return pltpu.make_async_copy(out_v[s], out_hbm.at[pl.ds(r0, CHUNK)], osem[s])

    def compute(ci, s):
        vv, gv, lv, ov = vals_v[s], gath_v[s], lens_v[s], out_v[s]

        def gbody(g, _):
            rvec = lax.iota(jnp.int32, LANES) * MAXLEN + g * (LANES * MAXLEN)
            acc = jnp.zeros((LANES,), jnp.float32)
            for l in range(MAXLEN):
                ivec = rvec + l
                acc = acc + plsc.load_gather(vv, [ivec]) * plsc.load_gather(gv, [ivec])
            sl = pl.ds(g * LANES, LANES)
            ov[sl] = acc / lv[sl]
            return ()

        lax.fori_loop(0, CHUNK // LANES, gbody, ())
        out_desc(ci, s).start()

    fire_in(0, 0)
    fire_in(1, 1)
    wait_in(0, 0)
    fire_g(0)

    def step(c, s, first, next_g, next_in):
        n = 1 - s
        if next_g:
            wait_in(c + 1, n)
            fire_g(n)
        drain_g(s)
        if not first:
            out_desc(c - 2, s).wait()
        compute(c, s)
        if next_in:
            fire_in(c + 2, s)

    step(0, 0, True, True, True)
    step(1, 1, True, True, True)

    def pair(k, _):
        c = 2 * k
        step(c, 0, False, True, True)
        step(c + 1, 1, False, True, True)
        return ()

    lax.fori_loop(1, (N_CHUNKS - 2) // 2, pair, ())
    step(N_CHUNKS - 2, 0, False, True, False)
    step(N_CHUNKS - 1, 1, False, False, False)
    out_desc(N_CHUNKS - 2, 0).wait()
    out_desc(N_CHUNKS - 1, 1).wait()


@jax.jit
def _sc_call(vals, idx, lens, table):
    mesh = plsc.VectorSubcoreMesh(
        core_axis_name="c", subcore_axis_name="s",
        num_cores=NC, num_subcores=NS)
    dv = lambda shape, dt: pltpu.VMEM(shape, dt)
    return pl.kernel(
        _sc_body,
        out_type=jax.ShapeDtypeStruct((ROWS,), jnp.float32),
        mesh=mesh,
        compiler_params=pltpu.CompilerParams(needs_layout_passes=False),
        scratch_types=[
            pltpu.VMEM_SHARED((1000002,), jnp.float32),
            dv((IN,), jnp.int32), dv((IN,), jnp.int32),
            dv((IN,), jnp.float32), dv((IN,), jnp.float32),
            dv((IN,), jnp.float32), dv((IN,), jnp.float32),
            dv((CHUNK,), jnp.float32), dv((CHUNK,), jnp.float32),
            dv((CHUNK,), jnp.float32), dv((CHUNK,), jnp.float32),
            pltpu.SemaphoreType.DMA, pltpu.SemaphoreType.DMA,
            pltpu.SemaphoreType.DMA, pltpu.SemaphoreType.DMA,
            pltpu.SemaphoreType.DMA, pltpu.SemaphoreType.DMA,
        ],
    )(vals, idx, lens, table)


def kernel(feature_values, feature_idx, seq_lens, weights_first_order):
    vals = feature_values.reshape(-1)
    idx = feature_idx.astype(jnp.int32).reshape(-1)
    lens = seq_lens.astype(jnp.float32).T.reshape(-1)
    table = weights_first_order.reshape(-1)
    out = _sc_call(vals, idx, lens, table)
    return out.reshape(FIELDS, BATCH).T


# speedup vs baseline: 194.5793x; 1.9552x over previous
import jax
import jax.numpy as jnp
from jax import lax
from jax.experimental import pallas as pl
from jax.experimental.pallas import tpu as pltpu
from jax.experimental.pallas import tpu_sc as plsc

FIELDS = 26
BATCH = 4096
MAXLEN = 20
ROWS = FIELDS * BATCH
TABLE = 1000002

NC, NS, LANES = 2, 16, 16
NW = NC * NS
ROWS_PER_W = ROWS // NW
CHUNK = 128
N_CHUNKS = ROWS_PER_W // CHUNK


def _sc_body(vals_hbm, idx_hbm, lens_hbm, table_hbm, out_hbm,
             table_sh,
             idx_v0, idx_v1, vals_v0, vals_v1, gath_v0, gath_v1,
             lens_v0, lens_v1, out_v0, out_v1,
             isem0, isem1, gsem0, gsem1, osem0, osem1):
    sid = lax.axis_index("s")
    row0 = (sid * NC + lax.axis_index("c")) * ROWS_PER_W

    @pl.when(sid == 0)
    def _():
        pltpu.sync_copy(table_hbm, table_sh)

    plsc.subcore_barrier()
    idx_v = (idx_v0, idx_v1)
    vals_v = (vals_v0, vals_v1)
    gath_v = (gath_v0, gath_v1)
    lens_v = (lens_v0, lens_v1)
    out_v = (out_v0, out_v1)
    isem = (isem0, isem1)
    gsem = (gsem0, gsem1)
    osem = (osem0, osem1)

    def in_descs(ci, s):
        r0 = row0 + ci * CHUNK
        sl = pl.ds(r0, CHUNK)
        return (
            pltpu.make_async_copy(idx_hbm.at[:, sl], idx_v[s], isem[s]),
            pltpu.make_async_copy(vals_hbm.at[:, sl], vals_v[s], isem[s]),
            pltpu.make_async_copy(lens_hbm.at[sl], lens_v[s], isem[s]),
        )

    def fire_in(ci, s):
        for d in in_descs(ci, s):
            d.start()

    def wait_in(ci, s):
        for d in in_descs(ci, s):
            d.wait()

    def g_descs(s):
        out = []
        for l in range(MAXLEN):
            out.append(pltpu.make_async_copy(
                table_sh.at[idx_v[s].at[l, :]], gath_v[s].at[l, :], gsem[s]))
        return out

    def fire_g(s):
        for d in g_descs(s):
            d.start()

    def drain_g(s):
        for d in g_descs(s):
            d.wait()

    def out_desc(ci, s):
        r0 = row0 + ci * CHUNK
        return pltpu.make_async_copy(out_v[s], out_hbm.at[pl.ds(r0, CHUNK)], osem[s])

    def compute(ci, s):
        vv, gv, lv, ov = vals_v[s], gath_v[s], lens_v[s], out_v[s]
        for g in range(CHUNK // LANES):
            sl = pl.ds(g * LANES, LANES)
            acc = jnp.zeros((LANES,), jnp.float32)
            for l in range(MAXLEN):
                acc = acc + vv[l, sl] * gv[l, sl]
            ov[sl] = acc / lv[sl].astype(jnp.float32)
        out_desc(ci, s).start()

    fire_in(0, 0)
    fire_in(1, 1)
    wait_in(0, 0)
    fire_g(0)

    def step(c, s, first, next_g, next_in):
        n = 1 - s
        if next_g:
            wait_in(c + 1, n)
            fire_g(n)
        drain_g(s)
        if not first:
            out_desc(c - 2, s).wait()
        compute(c, s)
        if next_in:
            fire_in(c + 2, s)

    step(0, 0, True, True, True)
    step(1, 1, True, True, True)

    def pair(k, _):
        c = 2 * k
        step(c, 0, False, True, True)
        step(c + 1, 1, False, True, True)
        return ()

    lax.fori_loop(1, (N_CHUNKS - 2) // 2, pair, ())
    step(N_CHUNKS - 2, 0, False, True, False)
    step(N_CHUNKS - 1, 1, False, False, False)
    out_desc(N_CHUNKS - 2, 0).wait()
    out_desc(N_CHUNKS - 1, 1).wait()


@jax.jit
def _sc_call(vals_t, idx_t, lens, table):
    mesh = plsc.VectorSubcoreMesh(
        core_axis_name="c", subcore_axis_name="s",
        num_cores=NC, num_subcores=NS)
    dv = lambda shape, dt: pltpu.VMEM(shape, dt)
    return pl.kernel(
        _sc_body,
        out_type=jax.ShapeDtypeStruct((ROWS,), jnp.float32),
        mesh=mesh,
        compiler_params=pltpu.CompilerParams(needs_layout_passes=False),
        scratch_types=[
            pltpu.VMEM_SHARED((TABLE,), jnp.float32),
            dv((MAXLEN, CHUNK), jnp.int32), dv((MAXLEN, CHUNK), jnp.int32),
            dv((MAXLEN, CHUNK), jnp.float32), dv((MAXLEN, CHUNK), jnp.float32),
            dv((MAXLEN, CHUNK), jnp.float32), dv((MAXLEN, CHUNK), jnp.float32),
            dv((CHUNK,), jnp.int32), dv((CHUNK,), jnp.int32),
            dv((CHUNK,), jnp.float32), dv((CHUNK,), jnp.float32),
            pltpu.SemaphoreType.DMA, pltpu.SemaphoreType.DMA,
            pltpu.SemaphoreType.DMA, pltpu.SemaphoreType.DMA,
            pltpu.SemaphoreType.DMA, pltpu.SemaphoreType.DMA,
        ],
    )(vals_t, idx_t, lens, table)


def kernel(feature_values, feature_idx, seq_lens, weights_first_order):
    vals_t = feature_values.T
    idx_t = feature_idx.astype(jnp.int32).T
    lens = seq_lens.T.reshape(-1)
    table = weights_first_order.reshape(-1)
    out = _sc_call(vals_t, idx_t, lens, table)
    return out.reshape(FIELDS, BATCH).T


# speedup vs baseline: 305.8745x; 1.5720x over previous
import jax
import jax.numpy as jnp
from jax import lax
from jax.experimental import pallas as pl
from jax.experimental.pallas import tpu as pltpu
from jax.experimental.pallas import tpu_sc as plsc

FIELDS = 26
BATCH = 4096
MAXLEN = 20
ROWS = FIELDS * BATCH
TABLE = 1000002

NC, NS, LANES = 2, 16, 16
NW = NC * NS
ROWS_PER_W = ROWS // NW
CHUNK = 128
N_CHUNKS = ROWS_PER_W // CHUNK


def _sc_body(vals_hbm, idx_hbm, lens_hbm, table_hbm, out_hbm,
             table_sh,
             idx_v0, idx_v1, vals_v0, vals_v1, gath_v0, gath_v1,
             lens_v0, lens_v1, out_v0, out_v1,
             isem0, isem1, gsem0, gsem1, osem0, osem1):
    sid = lax.axis_index("s")
    row0 = (sid * NC + lax.axis_index("c")) * ROWS_PER_W

    @pl.when(sid == 0)
    def _():
        pltpu.sync_copy(table_hbm.at[0, :], table_sh)

    plsc.subcore_barrier()
    idx_v = (idx_v0, idx_v1)
    vals_v = (vals_v0, vals_v1)
    gath_v = (gath_v0, gath_v1)
    lens_v = (lens_v0, lens_v1)
    out_v = (out_v0, out_v1)
    isem = (isem0, isem1)
    gsem = (gsem0, gsem1)
    osem = (osem0, osem1)

    def in_descs(ci, s):
        r0 = row0 + ci * CHUNK
        sl = pl.ds(r0, CHUNK)
        return (
            pltpu.make_async_copy(idx_hbm.at[:, sl], idx_v[s], isem[s]),
            pltpu.make_async_copy(vals_hbm.at[:, sl], vals_v[s], isem[s]),
            pltpu.make_async_copy(lens_hbm.at[sl], lens_v[s], isem[s]),
        )

    def fire_in(ci, s):
        for d in in_descs(ci, s):
            d.start()

    def wait_in(ci, s):
        for d in in_descs(ci, s):
            d.wait()

    def g_descs(s):
        out = []
        for l in range(MAXLEN):
            out.append(pltpu.make_async_copy(
                table_sh.at[idx_v[s].at[l, :]], gath_v[s].at[l, :], gsem[s]))
        return out

    def fire_g(s):
        for d in g_descs(s):
            d.start()

    def drain_g(s):
        for d in g_descs(s):
            d.wait()

    def out_desc(ci, s):
        r0 = row0 + ci * CHUNK
        return pltpu.make_async_copy(out_v[s], out_hbm.at[pl.ds(r0, CHUNK)], osem[s])

    def compute(ci, s):
        vv, gv, lv, ov = vals_v[s], gath_v[s], lens_v[s], out_v[s]
        for g in range(CHUNK // LANES):
            sl = pl.ds(g * LANES, LANES)
            acc = jnp.zeros((LANES,), jnp.float32)
            for l in range(MAXLEN):
                acc = acc + vv[l, sl] * gv[l, sl]
            ov[sl] = acc / lv[sl].astype(jnp.float32)
        out_desc(ci, s).start()

    fire_in(0, 0)
    fire_in(1, 1)
    wait_in(0, 0)
    fire_g(0)

    def step(c, s, first, next_g, next_in):
        n = 1 - s
        if next_g:
            wait_in(c + 1, n)
            fire_g(n)
        drain_g(s)
        if not first:
            out_desc(c - 2, s).wait()
        compute(c, s)
        if next_in:
            fire_in(c + 2, s)

    step(0, 0, True, True, True)
    step(1, 1, True, True, True)

    def pair(k, _):
        c = 2 * k
        step(c, 0, False, True, True)
        step(c + 1, 1, False, True, True)
        return ()

    lax.fori_loop(1, (N_CHUNKS - 2) // 2, pair, ())
    step(N_CHUNKS - 2, 0, False, True, False)
    step(N_CHUNKS - 1, 1, False, False, False)
    out_desc(N_CHUNKS - 2, 0).wait()
    out_desc(N_CHUNKS - 1, 1).wait()


@jax.jit
def _sc_call(vals_t, idx_t, lens, table):
    mesh = plsc.VectorSubcoreMesh(
        core_axis_name="c", subcore_axis_name="s",
        num_cores=NC, num_subcores=NS)
    dv = lambda shape, dt: pltpu.VMEM(shape, dt)
    return pl.kernel(
        _sc_body,
        out_type=jax.ShapeDtypeStruct((ROWS,), jnp.float32),
        mesh=mesh,
        compiler_params=pltpu.CompilerParams(needs_layout_passes=False),
        scratch_types=[
            pltpu.VMEM_SHARED((TABLE,), jnp.float32),
            dv((MAXLEN, CHUNK), jnp.int32), dv((MAXLEN, CHUNK), jnp.int32),
            dv((MAXLEN, CHUNK), jnp.float32), dv((MAXLEN, CHUNK), jnp.float32),
            dv((MAXLEN, CHUNK), jnp.float32), dv((MAXLEN, CHUNK), jnp.float32),
            dv((CHUNK,), jnp.int32), dv((CHUNK,), jnp.int32),
            dv((CHUNK,), jnp.float32), dv((CHUNK,), jnp.float32),
            pltpu.SemaphoreType.DMA, pltpu.SemaphoreType.DMA,
            pltpu.SemaphoreType.DMA, pltpu.SemaphoreType.DMA,
            pltpu.SemaphoreType.DMA, pltpu.SemaphoreType.DMA,
        ],
    )(vals_t, idx_t, lens, table)


def kernel(feature_values, feature_idx, seq_lens, weights_first_order):
    vals_t = feature_values.T
    idx_t = feature_idx.astype(jnp.int32).T
    lens = seq_lens.T.reshape(-1)
    table = weights_first_order.T
    out = _sc_call(vals_t, idx_t, lens, table)
    return out.reshape(FIELDS, BATCH).T


# speedup vs baseline: 306.2761x; 1.0013x over previous
import jax
import jax.numpy as jnp
from jax import lax
from jax.experimental import pallas as pl
from jax.experimental.pallas import tpu as pltpu
from jax.experimental.pallas import tpu_sc as plsc

FIELDS = 26
BATCH = 4096
MAXLEN = 20
ROWS = FIELDS * BATCH
TABLE = 1000002

NC, NS, LANES = 2, 16, 16
NW = NC * NS
ROWS_PER_W = ROWS // NW
CHUNK = 128
N_CHUNKS = ROWS_PER_W // CHUNK


def _sc_body(vals_hbm, idx_hbm, lens_hbm, table_hbm, out_hbm,
             table_sh,
             idx_v0, idx_v1, vals_v0, vals_v1, gath_v0, gath_v1,
             lens_v0, lens_v1, out_v0, out_v1,
             isem0, isem1, gsem0, gsem1, osem0, osem1):
    sid = lax.axis_index("s")
    row0 = (sid * NC + lax.axis_index("c")) * ROWS_PER_W

    @pl.when(sid == 0)
    def _():
        pltpu.sync_copy(table_hbm.at[0, :], table_sh)

    plsc.subcore_barrier()
    idx_v = (idx_v0, idx_v1)
    vals_v = (vals_v0, vals_v1)
    gath_v = (gath_v0, gath_v1)
    lens_v = (lens_v0, lens_v1)
    out_v = (out_v0, out_v1)
    isem = (isem0, isem1)
    gsem = (gsem0, gsem1)
    osem = (osem0, osem1)

    def in_descs(ci, s):
        r0 = row0 + ci * CHUNK
        sl = pl.ds(r0, CHUNK)
        return (
            pltpu.make_async_copy(idx_hbm.at[:, sl], idx_v[s], isem[s]),
            pltpu.make_async_copy(vals_hbm.at[:, sl], vals_v[s], isem[s]),
            pltpu.make_async_copy(lens_hbm.at[sl], lens_v[s], isem[s]),
        )

    def fire_in(ci, s):
        for d in in_descs(ci, s):
            d.start()

    def wait_in(ci, s):
        for d in in_descs(ci, s):
            d.wait()

    def g_descs(s):
        out = []
        for l in range(MAXLEN):
            out.append(pltpu.make_async_copy(
                table_sh.at[idx_v[s].at[l, :]], gath_v[s].at[l, :], gsem[s]))
        return out

    def fire_g(s):
        for d in g_descs(s):
            d.start()

    def drain_g(s):
        for d in g_descs(s):
            d.wait()

    def out_desc(ci, s):
        r0 = row0 + ci * CHUNK
        return pltpu.make_async_copy(out_v[s], out_hbm.at[pl.ds(r0, CHUNK)], osem[s])

    def compute(ci, s):
        vv, gv, lv, ov = vals_v[s], gath_v[s], lens_v[s], out_v[s]
        for g in range(CHUNK // LANES):
            sl = pl.ds(g * LANES, LANES)
            acc = jnp.zeros((LANES,), jnp.float32)
            for l in range(MAXLEN):
                acc = acc + vv[l, sl] * gv[l, sl]
            ov[sl] = acc / lv[sl].astype(jnp.float32)
        out_desc(ci, s).start()

    fire_in(0, 0)
    fire_in(1, 1)
    wait_in(0, 0)
    fire_g(0)

    def step(c, s, first, next_g, next_in):
        n = 1 - s
        if next_g:
            wait_in(c + 1, n)
            fire_g(n)
        drain_g(s)
        if not first:
            out_desc(c - 2, s).wait()
        compute(c, s)
        if next_in:
            fire_in(c + 2, s)

    step(0, 0, True, True, True)
    step(1, 1, True, True, True)

    def pair(k, _):
        c = 2 * k
        step(c, 0, False, True, True)
        step(c + 1, 1, False, True, True)
        return ()

    if N_CHUNKS % 2 == 0:
        lax.fori_loop(1, (N_CHUNKS - 2) // 2, pair, ())
        step(N_CHUNKS - 2, 0, False, True, False)
        step(N_CHUNKS - 1, 1, False, False, False)
        out_desc(N_CHUNKS - 2, 0).wait()
        out_desc(N_CHUNKS - 1, 1).wait()
    else:
        lax.fori_loop(1, (N_CHUNKS - 3) // 2, pair, ())
        step(N_CHUNKS - 3, 0, False, True, True)
        step(N_CHUNKS - 2, 1, False, True, False)
        step(N_CHUNKS - 1, 0, False, False, False)
        out_desc(N_CHUNKS - 2, 1).wait()
        out_desc(N_CHUNKS - 1, 0).wait()


@jax.jit
def _sc_call(vals_t, idx_t, lens, table):
    mesh = plsc.VectorSubcoreMesh(
        core_axis_name="c", subcore_axis_name="s",
        num_cores=NC, num_subcores=NS)
    dv = lambda shape, dt: pltpu.VMEM(shape, dt)
    return pl.kernel(
        _sc_body,
        out_type=jax.ShapeDtypeStruct((ROWS,), jnp.float32),
        mesh=mesh,
        compiler_params=pltpu.CompilerParams(needs_layout_passes=False),
        scratch_types=[
            pltpu.VMEM_SHARED((TABLE,), jnp.float32),
            dv((MAXLEN, CHUNK), jnp.int32), dv((MAXLEN, CHUNK), jnp.int32),
            dv((MAXLEN, CHUNK), jnp.float32), dv((MAXLEN, CHUNK), jnp.float32),
            dv((MAXLEN, CHUNK), jnp.float32), dv((MAXLEN, CHUNK), jnp.float32),
            dv((CHUNK,), jnp.int32), dv((CHUNK,), jnp.int32),
            dv((CHUNK,), jnp.float32), dv((CHUNK,), jnp.float32),
            pltpu.SemaphoreType.DMA, pltpu.SemaphoreType.DMA,
            pltpu.SemaphoreType.DMA, pltpu.SemaphoreType.DMA,
            pltpu.SemaphoreType.DMA, pltpu.SemaphoreType.DMA,
        ],
    )(vals_t, idx_t, lens, table)


def kernel(feature_values, feature_idx, seq_lens, weights_first_order):
    vals_t = feature_values.T
    idx_t = feature_idx.astype(jnp.int32).T
    lens = seq_lens.T.reshape(-1)
    table = weights_first_order.T
    out = _sc_call(vals_t, idx_t, lens, table)
    return out.reshape(FIELDS, BATCH).T
